# baseline (speedup 1.0000x reference)
import jax
import jax.numpy as jnp
from jax.experimental import pallas as pl
from jax.experimental.pallas import tpu as pltpu

QBINS = 16
TILE_D = 1024


def _encoder_body(x_ref, rand_ref, id_ref, lvl_ref, out_ref, m_ref):
    dim_in = x_ref.shape[1]

    @pl.when(pl.program_id(0) == 0)
    def _build_onehot():
        xc = jnp.clip(x_ref[...], -1.0, 1.0)
        idxf = jnp.ceil(xc * 8.0) + 8.0
        for q in range(QBINS + 1):
            m_ref[:, q * dim_in : (q + 1) * dim_in] = (
                (idxf == float(q)).astype(jnp.float8_e4m3fn)
            )

    id_i8 = jax.lax.bitcast_convert_type(
        id_ref[...].astype(jnp.float8_e4m3fn), jnp.int8
    )
    lvl_sign = jax.lax.bitcast_convert_type(
        lvl_ref[...].astype(jnp.float8_e4m3fn), jnp.int8
    ) & jnp.int8(-128)

    acc = jnp.zeros(out_ref.shape, jnp.float32)
    for q in range(QBINS + 1):
        m_q = m_ref[:, q * dim_in : (q + 1) * dim_in]
        w_q = jax.lax.bitcast_convert_type(
            id_i8 ^ lvl_sign[q : q + 1, :], jnp.float8_e4m3fn
        )
        acc += jax.lax.dot_general(
            m_q, w_q,
            dimension_numbers=(((1,), (0,)), ((), ())),
            preferred_element_type=jnp.float32,
        )

    enc = jnp.clip(acc, -1.0, 1.0)
    signs = jnp.where(rand_ref[...] < 0.5, 1.0, -1.0)
    out_ref[...] = jnp.where(enc == 0.0, signs, enc)


def kernel(x, rand_u, id_hvs, lvl_hvs, intervals):
    del intervals
    batch, dim_in = x.shape
    d = id_hvs.shape[1]
    n_tiles = d // TILE_D
    qp1 = lvl_hvs.shape[0]
    return pl.pallas_call(
        _encoder_body,
        grid=(n_tiles,),
        in_specs=[
            pl.BlockSpec((batch, dim_in), lambda j: (0, 0)),
            pl.BlockSpec((batch, TILE_D), lambda j: (0, j)),
            pl.BlockSpec((dim_in, TILE_D), lambda j: (0, j)),
            pl.BlockSpec((qp1, TILE_D), lambda j: (0, j)),
        ],
        out_specs=pl.BlockSpec((batch, TILE_D), lambda j: (0, j)),
        out_shape=jax.ShapeDtypeStruct((batch, d), jnp.float32),
        scratch_shapes=[
            pltpu.VMEM((batch, (QBINS + 1) * dim_in), jnp.float8_e4m3fn)
        ],
    )(x, rand_u, id_hvs, lvl_hvs)


# speedup vs baseline: 1.0110x; 1.0110x over previous
import jax
import jax.numpy as jnp
from jax.experimental import pallas as pl
from jax.experimental.pallas import tpu as pltpu

QBINS = 16
TILE_D = 1024


def _encoder_body(x_ref, rand_ref, id_ref, lvl_ref, out_ref):
    xc = jnp.clip(x_ref[...], -1.0, 1.0)
    idxf = jnp.ceil(xc * 8.0) + 8.0

    id_i8 = jax.lax.bitcast_convert_type(
        id_ref[...].astype(jnp.float8_e4m3fn), jnp.int8
    )
    lvl_sign = jax.lax.bitcast_convert_type(
        lvl_ref[...].astype(jnp.float8_e4m3fn), jnp.int8
    ) & jnp.int8(-128)

    acc = jnp.zeros(out_ref.shape, jnp.float32)
    for q in range(QBINS + 1):
        m_q = (idxf == float(q)).astype(jnp.float8_e4m3fn)
        w_q = jax.lax.bitcast_convert_type(
            id_i8 ^ lvl_sign[q : q + 1, :], jnp.float8_e4m3fn
        )
        acc += jax.lax.dot_general(
            m_q, w_q,
            dimension_numbers=(((1,), (0,)), ((), ())),
            preferred_element_type=jnp.float32,
        )

    enc = jnp.clip(acc, -1.0, 1.0)
    signs = jnp.where(rand_ref[...] < 0.5, 1.0, -1.0)
    out_ref[...] = jnp.where(enc == 0.0, signs, enc)


def kernel(x, rand_u, id_hvs, lvl_hvs, intervals):
    del intervals
    batch, dim_in = x.shape
    d = id_hvs.shape[1]
    n_tiles = d // TILE_D
    qp1 = lvl_hvs.shape[0]
    return pl.pallas_call(
        _encoder_body,
        grid=(n_tiles,),
        in_specs=[
            pl.BlockSpec((batch, dim_in), lambda j: (0, 0)),
            pl.BlockSpec((batch, TILE_D), lambda j: (0, j)),
            pl.BlockSpec((dim_in, TILE_D), lambda j: (0, j)),
            pl.BlockSpec((qp1, TILE_D), lambda j: (0, j)),
        ],
        out_specs=pl.BlockSpec((batch, TILE_D), lambda j: (0, j)),
        out_shape=jax.ShapeDtypeStruct((batch, d), jnp.float32),
    )(x, rand_u, id_hvs, lvl_hvs)
